# baseline (speedup 1.0000x reference)
import functools

import jax
import jax.numpy as jnp
from jax.experimental import pallas as pl
from jax.experimental.pallas import tpu as pltpu

_EPS = 1e-7
_NBUF = 8


def _ctc_body(hbm_ref, dec_ref, sc_ref, vbuf, sems, bestcols, widx):
    B, T, C = hbm_ref.shape

    def start(i):
        buf = jax.lax.rem(i, _NBUF)
        pltpu.make_async_copy(hbm_ref.at[i], vbuf.at[buf], sems.at[buf]).start()

    def wait(i):
        buf = jax.lax.rem(i, _NBUF)
        pltpu.make_async_copy(hbm_ref.at[i], vbuf.at[buf], sems.at[buf]).wait()

    for k in range(_NBUF):
        start(k)

    lane = jax.lax.broadcasted_iota(jnp.int32, (T, C), 1)
    blane = jax.lax.broadcasted_iota(jnp.int32, (T, B), 1)
    slane = jax.lax.broadcasted_iota(jnp.int32, (1, B), 1)

    def loop(b, ssum):
        wait(b)
        buf = jax.lax.rem(b, _NBUF)
        x = vbuf[buf]

        top1 = jnp.max(x, axis=1, keepdims=True)
        m1 = x == top1
        idx1 = jnp.min(jnp.where(m1, lane, C), axis=1, keepdims=True)
        top2 = jnp.max(jnp.where(m1, -jnp.inf, x), axis=1, keepdims=True)

        log_top1 = jnp.log(top1 + _EPS)
        collide = jnp.log(top2 + _EPS) == log_top1

        widx[...] = idx1

        CH = 64
        for ck in range(T // CH):
            lo, hi = ck * CH, (ck + 1) * CH
            ccol = collide[lo:hi]

            @pl.when(jnp.any(ccol))
            def _(lo=lo, hi=hi, ccol=ccol):
                logx = jnp.log(x[lo:hi] + _EPS)
                wc = jnp.min(
                    jnp.where(logx == log_top1[lo:hi], lane[:CH], C),
                    axis=1, keepdims=True)
                widx[lo:hi] = jnp.where(ccol, wc, idx1[lo:hi])

        winner = widx[...]
        bestcols[...] = jnp.where(blane == b, winner, bestcols[...])

        @pl.when(b + _NBUF < B)
        def _():
            start(b + _NBUF)

        return jnp.where(slane == b, -jnp.sum(log_top1), ssum)

    ssum = jax.lax.fori_loop(0, B, loop, jnp.zeros((1, B), jnp.float32))
    sc_ref[...] = ssum

    bb = bestcols[...]
    blank_val = C - 1
    sub = jax.lax.broadcasted_iota(jnp.int32, (T, B), 0)

    prev = pltpu.roll(bb, 1, axis=0)
    prev = jnp.where(sub == 0, -1, prev)
    keep = (bb != prev) & (bb != blank_val)

    c = keep.astype(jnp.int32)
    sh = 1
    while sh < T:
        c = c + jnp.where(sub >= sh, pltpu.roll(c, sh, axis=0), 0)
        sh *= 2

    posn = c - 1
    v = jnp.where(keep, bb, -1)
    s = jnp.where(keep, sub - posn, 0)

    k = 0
    sh = 1
    while sh < T:
        cand_v = pltpu.roll(v, T - sh, axis=0)
        cand_s = pltpu.roll(s, T - sh, axis=0)
        valid = sub < T - sh
        take = valid & (cand_v >= 0) & (((cand_s >> k) & 1) == 1)
        stay = (v >= 0) & (((s >> k) & 1) == 0)
        v = jnp.where(take, cand_v, jnp.where(stay, v, -1))
        s = jnp.where(take, cand_s - sh, jnp.where(stay, s, 0))
        k += 1
        sh *= 2

    dec_ref[...] = jnp.transpose(v)


def kernel(inputs):
    B, T, C = inputs.shape
    dec, scores_row = pl.pallas_call(
        _ctc_body,
        in_specs=[pl.BlockSpec(memory_space=pltpu.HBM)],
        out_specs=[
            pl.BlockSpec(memory_space=pltpu.VMEM),
            pl.BlockSpec(memory_space=pltpu.VMEM),
        ],
        out_shape=[
            jax.ShapeDtypeStruct((B, T), jnp.int32),
            jax.ShapeDtypeStruct((1, B), jnp.float32),
        ],
        scratch_shapes=[
            pltpu.VMEM((_NBUF, T, C), jnp.float32),
            pltpu.SemaphoreType.DMA((_NBUF,)),
            pltpu.VMEM((T, B), jnp.int32),
            pltpu.VMEM((T, 1), jnp.int32),
        ],
    )(inputs)
    return dec, scores_row.reshape(B, 1)


# speedup vs baseline: 1.0936x; 1.0936x over previous
import functools

import jax
import jax.numpy as jnp
from jax.experimental import pallas as pl
from jax.experimental.pallas import tpu as pltpu

_EPS = 1e-7
_NBUF = 8


def _ctc_body(hbm_ref, dec_ref, sc_ref, vbuf, sems, bestcols, widx):
    B, T, C = hbm_ref.shape

    def start(i):
        buf = jax.lax.rem(i, _NBUF)
        pltpu.make_async_copy(hbm_ref.at[i], vbuf.at[buf], sems.at[buf]).start()

    def wait(i):
        buf = jax.lax.rem(i, _NBUF)
        pltpu.make_async_copy(hbm_ref.at[i], vbuf.at[buf], sems.at[buf]).wait()

    for k in range(_NBUF):
        start(k)

    lane = jax.lax.broadcasted_iota(jnp.int32, (T, C), 1)
    blane = jax.lax.broadcasted_iota(jnp.int32, (T, B), 1)
    slane = jax.lax.broadcasted_iota(jnp.int32, (1, B), 1)

    def loop(b, ssum):
        wait(b)
        buf = jax.lax.rem(b, _NBUF)
        x = vbuf[buf]

        top1 = jnp.max(x, axis=1, keepdims=True)
        m1 = x == top1
        idx1 = jnp.min(jnp.where(m1, lane, C), axis=1, keepdims=True)
        top2 = jnp.max(jnp.where(m1, -jnp.inf, x), axis=1, keepdims=True)

        log_top1 = jnp.log(top1 + _EPS)
        collide = jnp.log(top2 + _EPS) == log_top1

        widx[...] = idx1

        @pl.when(jnp.any(collide))
        def _():
            logx = jnp.log(x + _EPS)
            wc = jnp.min(jnp.where(logx == log_top1, lane, C), axis=1,
                         keepdims=True)
            widx[...] = jnp.where(collide, wc, idx1)

        winner = widx[...]
        bestcols[...] = jnp.where(blane == b, winner, bestcols[...])

        @pl.when(b + _NBUF < B)
        def _():
            start(b + _NBUF)

        return jnp.where(slane == b, -jnp.sum(log_top1), ssum)

    ssum = jax.lax.fori_loop(0, B, loop, jnp.zeros((1, B), jnp.float32))
    sc_ref[...] = ssum

    bb = bestcols[...]
    blank_val = C - 1
    sub = jax.lax.broadcasted_iota(jnp.int32, (T, B), 0)

    prev = pltpu.roll(bb, 1, axis=0)
    prev = jnp.where(sub == 0, -1, prev)
    keep = (bb != prev) & (bb != blank_val)

    c = keep.astype(jnp.int32)
    sh = 1
    while sh < T:
        c = c + jnp.where(sub >= sh, pltpu.roll(c, sh, axis=0), 0)
        sh *= 2

    posn = c - 1
    v = jnp.where(keep, bb, -1)
    s = jnp.where(keep, sub - posn, 0)

    k = 0
    sh = 1
    while sh < T:
        cand_v = pltpu.roll(v, T - sh, axis=0)
        cand_s = pltpu.roll(s, T - sh, axis=0)
        valid = sub < T - sh
        take = valid & (cand_v >= 0) & (((cand_s >> k) & 1) == 1)
        stay = (v >= 0) & (((s >> k) & 1) == 0)
        v = jnp.where(take, cand_v, jnp.where(stay, v, -1))
        s = jnp.where(take, cand_s - sh, jnp.where(stay, s, 0))
        k += 1
        sh *= 2

    dec_ref[...] = jnp.transpose(v)


def kernel(inputs):
    B, T, C = inputs.shape
    dec, scores_row = pl.pallas_call(
        _ctc_body,
        in_specs=[pl.BlockSpec(memory_space=pltpu.HBM)],
        out_specs=[
            pl.BlockSpec(memory_space=pltpu.VMEM),
            pl.BlockSpec(memory_space=pltpu.VMEM),
        ],
        out_shape=[
            jax.ShapeDtypeStruct((B, T), jnp.int32),
            jax.ShapeDtypeStruct((1, B), jnp.float32),
        ],
        scratch_shapes=[
            pltpu.VMEM((_NBUF, T, C), jnp.float32),
            pltpu.SemaphoreType.DMA((_NBUF,)),
            pltpu.VMEM((T, B), jnp.int32),
            pltpu.VMEM((T, 1), jnp.int32),
        ],
    )(inputs)
    return dec, scores_row.reshape(B, 1)


# speedup vs baseline: 1.0950x; 1.0013x over previous
import jax
import jax.numpy as jnp
from jax.experimental import pallas as pl
from jax.experimental.pallas import tpu as pltpu

_EPS = 1e-7
_NBUF = 8


def _ctc_body(hbm_ref, dec_ref, sc_ref, vbuf, sems, bestcols, widx):
    B, T, C = hbm_ref.shape

    def start(i):
        buf = jax.lax.rem(i, _NBUF)
        pltpu.make_async_copy(hbm_ref.at[i], vbuf.at[buf], sems.at[buf]).start()

    def wait(i):
        buf = jax.lax.rem(i, _NBUF)
        pltpu.make_async_copy(hbm_ref.at[i], vbuf.at[buf], sems.at[buf]).wait()

    for k in range(_NBUF):
        start(k)

    lane = jax.lax.broadcasted_iota(jnp.int32, (T, C), 1)
    blane = jax.lax.broadcasted_iota(jnp.int32, (T, B), 1)
    slane = jax.lax.broadcasted_iota(jnp.int32, (1, B), 1)

    def loop(b, ssum):
        wait(b)
        buf = jax.lax.rem(b, _NBUF)
        x = vbuf[buf]

        top1 = jnp.max(x, axis=1, keepdims=True)
        m1 = x == top1
        idx1 = jnp.min(jnp.where(m1, lane, C), axis=1, keepdims=True)
        top2 = jnp.max(jnp.where(m1, -jnp.inf, x), axis=1, keepdims=True)

        log_top1 = jnp.log(top1 + _EPS)
        collide = jnp.log(top2 + _EPS) == log_top1

        widx[...] = idx1

        @pl.when(jnp.any(collide))
        def _():
            logx = jnp.log(x + _EPS)
            wc = jnp.min(jnp.where(logx == log_top1, lane, C), axis=1,
                         keepdims=True)
            widx[...] = jnp.where(collide, wc, idx1)

        winner = widx[...]
        bestcols[...] = jnp.where(blane == b, winner, bestcols[...])

        @pl.when(b + _NBUF < B)
        def _():
            start(b + _NBUF)

        return jnp.where(slane == b, -jnp.sum(log_top1), ssum)

    ssum = jax.lax.fori_loop(0, B, loop, jnp.zeros((1, B), jnp.float32))
    sc_ref[...] = ssum

    bb = bestcols[...]
    blank_val = C - 1
    sub = jax.lax.broadcasted_iota(jnp.int32, (T, B), 0)

    prev = pltpu.roll(bb, 1, axis=0)
    prev = jnp.where(sub == 0, -1, prev)
    keep = (bb != prev) & (bb != blank_val)

    c = keep.astype(jnp.int32)
    sh = 1
    while sh < T:
        c = c + jnp.where(sub >= sh, pltpu.roll(c, sh, axis=0), 0)
        sh *= 2

    posn = c - 1
    v = jnp.where(keep, bb, -1)
    s = jnp.where(keep, sub - posn, 0)

    k = 0
    sh = 1
    while sh < T:
        cand_v = pltpu.roll(v, T - sh, axis=0)
        cand_s = pltpu.roll(s, T - sh, axis=0)
        valid = sub < T - sh
        take = valid & (cand_v >= 0) & (((cand_s >> k) & 1) == 1)
        stay = (v >= 0) & (((s >> k) & 1) == 0)
        v = jnp.where(take, cand_v, jnp.where(stay, v, -1))
        s = jnp.where(take, cand_s - sh, jnp.where(stay, s, 0))
        k += 1
        sh *= 2

    dec_ref[...] = jnp.transpose(v)


def kernel(inputs):
    B, T, C = inputs.shape
    dec, scores_row = pl.pallas_call(
        _ctc_body,
        in_specs=[pl.BlockSpec(memory_space=pltpu.HBM)],
        out_specs=[
            pl.BlockSpec(memory_space=pltpu.VMEM),
            pl.BlockSpec(memory_space=pltpu.VMEM),
        ],
        out_shape=[
            jax.ShapeDtypeStruct((B, T), jnp.int32),
            jax.ShapeDtypeStruct((1, B), jnp.float32),
        ],
        scratch_shapes=[
            pltpu.VMEM((_NBUF, T, C), jnp.float32),
            pltpu.SemaphoreType.DMA((_NBUF,)),
            pltpu.VMEM((T, B), jnp.int32),
            pltpu.VMEM((T, 1), jnp.int32),
        ],
    )(inputs)
    return dec, scores_row.reshape(B, 1)
